# baseline (speedup 1.0000x reference)
import functools
import math

import jax
import jax.numpy as jnp
from jax import lax
from jax.experimental import pallas as pl
from jax.experimental.pallas import tpu as pltpu
from jax.experimental.pallas import tpu_sc as plsc

_C = 10
_B = 2048
_EPS = 0.05
_ROWS = 256
_K0 = 1.0 + math.exp(2.0 * _EPS)
_LN2 = math.log(2.0)
_NC = 2
_NS = 16
_NW = _NC * _NS
_CHUNK = _B // _NW


@functools.partial(
    pl.kernel,
    mesh=plsc.VectorSubcoreMesh(core_axis_name="c", subcore_axis_name="s"),
    out_type=[
        jax.ShapeDtypeStruct((_C, _B), jnp.float32),
        jax.ShapeDtypeStruct((_C, _B), jnp.float32),
        jax.ShapeDtypeStruct((_C, _B), jnp.float32),
        jax.ShapeDtypeStruct((_C, _B), jnp.float32),
        jax.ShapeDtypeStruct((_B,), jnp.float32),
        jax.ShapeDtypeStruct((_B,), jnp.float32),
        jax.ShapeDtypeStruct((_B,), jnp.float32),
        jax.ShapeDtypeStruct((_B,), jnp.float32),
    ],
    scratch_types=[
        pltpu.VMEM((_C, _CHUNK), jnp.float32),
        pltpu.VMEM((_C, _CHUNK), jnp.float32),
        pltpu.VMEM((_CHUNK,), jnp.int32),
        pltpu.VMEM((_C, _CHUNK), jnp.float32),
        pltpu.VMEM((_C, _CHUNK), jnp.float32),
        pltpu.VMEM((_C, _CHUNK), jnp.float32),
        pltpu.VMEM((_C, _CHUNK), jnp.float32),
        pltpu.VMEM((_CHUNK,), jnp.float32),
        pltpu.VMEM((_CHUNK,), jnp.float32),
        pltpu.VMEM((_CHUNK,), jnp.float32),
        pltpu.VMEM((_CHUNK,), jnp.float32),
    ],
)
def _sc_row_stats(ysT_hbm, ysaT_hbm, lab_hbm,
                  e4_hbm, e4i_hbm, t2_hbm, t2i_hbm,
                  ce_hbm, cei_hbm, cs_hbm, csi_hbm,
                  ys_v, ysa_v, lab_v,
                  e4_v, e4i_v, t2_v, t2i_v,
                  ce_v, cei_v, cs_v, csi_v):
    wid = lax.axis_index("s") * _NC + lax.axis_index("c")
    base = wid * _CHUNK
    csl = pl.ds(base, _CHUNK)

    for c in range(_C):
        pltpu.sync_copy(ysT_hbm.at[c, csl], ys_v.at[c])
        pltpu.sync_copy(ysaT_hbm.at[c, csl], ysa_v.at[c])
    pltpu.sync_copy(lab_hbm.at[csl], lab_v)

    def softmax_cols(v_ref, sl, lab_j):
        cols = [v_ref[c, sl] for c in range(_C)]
        m = cols[0]
        for c in range(1, _C):
            m = jnp.maximum(m, cols[c])
        es = [jnp.exp(cols[c] - m) for c in range(_C)]
        s = es[0]
        for c in range(1, _C):
            s = s + es[c]
        rs = 1.0 / s
        probs = [es[c] * rs for c in range(_C)]
        pick = jnp.zeros((16,), jnp.float32)
        for c in range(_C):
            pick = jnp.where(lab_j == c, probs[c], pick)
        return probs, pick

    for j in range(_CHUNK // 16):
        sl = pl.ds(16 * j, 16)
        lab_j = lab_v[sl]
        p, g = softmax_cols(ys_v, sl, lab_j)
        pa, ga = softmax_cols(ysa_v, sl, lab_j)
        for c in range(_C):
            e4 = jnp.exp(4.0 * p[c])
            t2 = jnp.exp(2.0 * (p[c] - pa[c]))
            e4_v[c, sl] = e4
            e4i_v[c, sl] = 1.0 / e4
            t2_v[c, sl] = t2
            t2i_v[c, sl] = 1.0 / t2
        ce_v[sl] = jnp.exp((_EPS + 4.0) - 4.0 * g)
        cei_v[sl] = jnp.exp((_EPS - 4.0) + 4.0 * g)
        cs_v[sl] = jnp.exp(_EPS + 2.0 * (ga - g))
        csi_v[sl] = jnp.exp(_EPS - 2.0 * (ga - g))

    for c in range(_C):
        pltpu.sync_copy(e4_v.at[c], e4_hbm.at[c, csl])
        pltpu.sync_copy(e4i_v.at[c], e4i_hbm.at[c, csl])
        pltpu.sync_copy(t2_v.at[c], t2_hbm.at[c, csl])
        pltpu.sync_copy(t2i_v.at[c], t2i_hbm.at[c, csl])
    pltpu.sync_copy(ce_v, ce_hbm.at[csl])
    pltpu.sync_copy(cei_v, cei_hbm.at[csl])
    pltpu.sync_copy(cs_v, cs_hbm.at[csl])
    pltpu.sync_copy(csi_v, csi_hbm.at[csl])


def _auc_kernel(e4_ref, e4i_ref, t2_ref, t2i_ref, ce_ref, cei_ref, cs_ref,
                csi_ref, labc_ref, labr_ref, emp_ref, disc_ref,
                w_ref, acc_e_ref, acc_s_ref):
    i = pl.program_id(0)
    nsteps = pl.num_programs(0)
    lab_row = labr_ref[...]

    @pl.when(i == 0)
    def _init():
        lab_all = labc_ref[...]
        w = jnp.zeros((_B, 1), jnp.float32)
        for c in range(_C):
            n_c = jnp.sum((lab_row == c).astype(jnp.float32))
            fac_c = _LN2 / (n_c * (_B - n_c))
            w = w + jnp.where(lab_all == c, fac_c, 0.0)
        w_ref[...] = w
        acc_e_ref[...] = jnp.zeros((1, _B), jnp.float32)
        acc_s_ref[...] = jnp.zeros((1, _B), jnp.float32)

    rows = pl.ds(i * _ROWS, _ROWS)
    lab_blk = labc_ref[rows, :]

    cls = jax.lax.broadcasted_iota(jnp.int32, (1, _C), 1)
    onehot = (lab_blk == cls).astype(jnp.float32)

    dot = functools.partial(
        jax.lax.dot_general,
        dimension_numbers=(((1,), (0,)), ((), ())),
        preferred_element_type=jnp.float32,
        precision=jax.lax.Precision.DEFAULT,
    )
    h_e = dot(onehot * ce_ref[rows, :], e4_ref[...])
    r_e = dot(onehot * cei_ref[rows, :], e4i_ref[...])
    h_s = dot(onehot * cs_ref[rows, :], t2_ref[...])
    r_s = dot(onehot * csi_ref[rows, :], t2i_ref[...])

    wv = jnp.where(lab_blk != lab_row, w_ref[rows, :], 0.0)

    l_e = jnp.log2(_K0 + h_e + r_e)
    l_s = jnp.log2(_K0 + h_s + r_s)
    acc_e_ref[...] += jnp.sum(wv * l_e, axis=0, keepdims=True)
    acc_s_ref[...] += jnp.sum(wv * l_s, axis=0, keepdims=True)

    @pl.when(i == nsteps - 1)
    def _finish():
        emp_ref[...] = jnp.sum(acc_e_ref[...]).reshape(1, 1)
        disc_ref[...] = jnp.sum(acc_s_ref[...]).reshape(1, 1)


def kernel(y_s, y_s_adv, labels_s, y_t, y_t_adv, epoch):
    lab = labels_s.astype(jnp.int32)
    e4, e4i, t2, t2i, ce, cei, cs, csi = _sc_row_stats(y_s.T, y_s_adv.T, lab)

    lab_col = lab.reshape(_B, 1)
    lab_row = lab.reshape(1, _B)

    grid = (_B // _ROWS,)
    emp, disc = pl.pallas_call(
        _auc_kernel,
        grid=grid,
        in_specs=[
            pl.BlockSpec((_C, _B), lambda i: (0, 0)),
            pl.BlockSpec((_C, _B), lambda i: (0, 0)),
            pl.BlockSpec((_C, _B), lambda i: (0, 0)),
            pl.BlockSpec((_C, _B), lambda i: (0, 0)),
            pl.BlockSpec((_B, 1), lambda i: (0, 0)),
            pl.BlockSpec((_B, 1), lambda i: (0, 0)),
            pl.BlockSpec((_B, 1), lambda i: (0, 0)),
            pl.BlockSpec((_B, 1), lambda i: (0, 0)),
            pl.BlockSpec((_B, 1), lambda i: (0, 0)),
            pl.BlockSpec((1, _B), lambda i: (0, 0)),
        ],
        out_specs=[
            pl.BlockSpec((1, 1), lambda i: (0, 0)),
            pl.BlockSpec((1, 1), lambda i: (0, 0)),
        ],
        out_shape=[
            jax.ShapeDtypeStruct((1, 1), jnp.float32),
            jax.ShapeDtypeStruct((1, 1), jnp.float32),
        ],
        scratch_shapes=[
            pltpu.VMEM((_B, 1), jnp.float32),
            pltpu.VMEM((1, _B), jnp.float32),
            pltpu.VMEM((1, _B), jnp.float32),
        ],
    )(e4, e4i, t2, t2i, ce.reshape(_B, 1), cei.reshape(_B, 1),
      cs.reshape(_B, 1), csi.reshape(_B, 1), lab_col, lab_row)

    empirical = 0.25 * emp[0, 0]
    transfer = -0.5 * disc[0, 0]
    return (empirical, transfer)


# speedup vs baseline: 1.1445x; 1.1445x over previous
import functools
import math

import jax
import jax.numpy as jnp
from jax import lax
from jax.experimental import pallas as pl
from jax.experimental.pallas import tpu as pltpu
from jax.experimental.pallas import tpu_sc as plsc

_C = 10
_B = 2048
_EPS = 0.05
_ROWS = 256
_K0 = 1.0 + math.exp(2.0 * _EPS)
_LN2 = math.log(2.0)
_NC = 2
_NS = 16
_ACT = 16
_CHUNK = _B // _ACT


@functools.partial(
    pl.kernel,
    mesh=plsc.VectorSubcoreMesh(core_axis_name="c", subcore_axis_name="s"),
    out_type=[
        jax.ShapeDtypeStruct((_C, _B), jnp.float32),
        jax.ShapeDtypeStruct((_C, _B), jnp.float32),
        jax.ShapeDtypeStruct((_C, _B), jnp.float32),
        jax.ShapeDtypeStruct((_C, _B), jnp.float32),
        jax.ShapeDtypeStruct((_B,), jnp.float32),
        jax.ShapeDtypeStruct((_B,), jnp.float32),
        jax.ShapeDtypeStruct((_B,), jnp.float32),
        jax.ShapeDtypeStruct((_B,), jnp.float32),
    ],
    scratch_types=[
        pltpu.VMEM((_C, _CHUNK), jnp.float32),
        pltpu.VMEM((_C, _CHUNK), jnp.float32),
        pltpu.VMEM((_CHUNK,), jnp.int32),
        pltpu.VMEM((_C, _CHUNK), jnp.float32),
        pltpu.VMEM((_C, _CHUNK), jnp.float32),
        pltpu.VMEM((_C, _CHUNK), jnp.float32),
        pltpu.VMEM((_C, _CHUNK), jnp.float32),
        pltpu.VMEM((_CHUNK,), jnp.float32),
        pltpu.VMEM((_CHUNK,), jnp.float32),
        pltpu.VMEM((_CHUNK,), jnp.float32),
        pltpu.VMEM((_CHUNK,), jnp.float32),
    ],
)
def _sc_row_stats(ysT_hbm, ysaT_hbm, lab_hbm,
                  e4_hbm, e4i_hbm, t2_hbm, t2i_hbm,
                  ce_hbm, cei_hbm, cs_hbm, csi_hbm,
                  ys_v, ysa_v, lab_v,
                  e4_v, e4i_v, t2_v, t2i_v,
                  ce_v, cei_v, cs_v, csi_v):
    wid = lax.axis_index("s") * _NC + lax.axis_index("c")

    @pl.when(wid < _ACT)
    def _body():
        base = wid * _CHUNK
        csl = pl.ds(base, _CHUNK)

        pltpu.sync_copy(ysT_hbm.at[:, csl], ys_v)
        pltpu.sync_copy(ysaT_hbm.at[:, csl], ysa_v)
        pltpu.sync_copy(lab_hbm.at[csl], lab_v)

        def softmax_cols(v_ref, sl, lab_j):
            cols = [v_ref[c, sl] for c in range(_C)]
            m = cols[0]
            for c in range(1, _C):
                m = jnp.maximum(m, cols[c])
            es = [jnp.exp(cols[c] - m) for c in range(_C)]
            s = es[0]
            for c in range(1, _C):
                s = s + es[c]
            rs = 1.0 / s
            probs = [es[c] * rs for c in range(_C)]
            pick = jnp.zeros((16,), jnp.float32)
            for c in range(_C):
                pick = jnp.where(lab_j == c, probs[c], pick)
            return probs, pick

        for j in range(_CHUNK // 16):
            sl = pl.ds(16 * j, 16)
            lab_j = lab_v[sl]
            p, g = softmax_cols(ys_v, sl, lab_j)
            pa, ga = softmax_cols(ysa_v, sl, lab_j)
            for c in range(_C):
                e4 = jnp.exp(4.0 * p[c])
                t2 = jnp.exp(2.0 * (p[c] - pa[c]))
                e4_v[c, sl] = e4
                e4i_v[c, sl] = 1.0 / e4
                t2_v[c, sl] = t2
                t2i_v[c, sl] = 1.0 / t2
            ce_v[sl] = jnp.exp((_EPS + 4.0) - 4.0 * g)
            cei_v[sl] = jnp.exp((_EPS - 4.0) + 4.0 * g)
            cs_v[sl] = jnp.exp(_EPS + 2.0 * (ga - g))
            csi_v[sl] = jnp.exp(_EPS - 2.0 * (ga - g))

        pltpu.sync_copy(e4_v, e4_hbm.at[:, csl])
        pltpu.sync_copy(e4i_v, e4i_hbm.at[:, csl])
        pltpu.sync_copy(t2_v, t2_hbm.at[:, csl])
        pltpu.sync_copy(t2i_v, t2i_hbm.at[:, csl])
        pltpu.sync_copy(ce_v, ce_hbm.at[csl])
        pltpu.sync_copy(cei_v, cei_hbm.at[csl])
        pltpu.sync_copy(cs_v, cs_hbm.at[csl])
        pltpu.sync_copy(csi_v, csi_hbm.at[csl])


def _auc_kernel(e4_ref, e4i_ref, t2_ref, t2i_ref, ce_ref, cei_ref, cs_ref,
                csi_ref, labc_ref, labr_ref, emp_ref, disc_ref,
                w_ref, acc_e_ref, acc_s_ref):
    i = pl.program_id(0)
    nsteps = pl.num_programs(0)
    lab_row = labr_ref[...]

    @pl.when(i == 0)
    def _init():
        lab_all = labc_ref[...]
        w = jnp.zeros((_B, 1), jnp.float32)
        for c in range(_C):
            n_c = jnp.sum((lab_row == c).astype(jnp.float32))
            fac_c = _LN2 / (n_c * (_B - n_c))
            w = w + jnp.where(lab_all == c, fac_c, 0.0)
        w_ref[...] = w
        acc_e_ref[...] = jnp.zeros((1, _B), jnp.float32)
        acc_s_ref[...] = jnp.zeros((1, _B), jnp.float32)

    rows = pl.ds(i * _ROWS, _ROWS)
    lab_blk = labc_ref[rows, :]

    cls = jax.lax.broadcasted_iota(jnp.int32, (1, _C), 1)
    onehot = (lab_blk == cls).astype(jnp.float32)

    dot = functools.partial(
        jax.lax.dot_general,
        dimension_numbers=(((1,), (0,)), ((), ())),
        preferred_element_type=jnp.float32,
        precision=jax.lax.Precision.DEFAULT,
    )
    h_e = dot(onehot * ce_ref[rows, :], e4_ref[...])
    r_e = dot(onehot * cei_ref[rows, :], e4i_ref[...])
    h_s = dot(onehot * cs_ref[rows, :], t2_ref[...])
    r_s = dot(onehot * csi_ref[rows, :], t2i_ref[...])

    wv = jnp.where(lab_blk != lab_row, w_ref[rows, :], 0.0)

    l_e = jnp.log2(_K0 + h_e + r_e)
    l_s = jnp.log2(_K0 + h_s + r_s)
    acc_e_ref[...] += jnp.sum(wv * l_e, axis=0, keepdims=True)
    acc_s_ref[...] += jnp.sum(wv * l_s, axis=0, keepdims=True)

    @pl.when(i == nsteps - 1)
    def _finish():
        emp_ref[...] = jnp.sum(acc_e_ref[...]).reshape(1, 1)
        disc_ref[...] = jnp.sum(acc_s_ref[...]).reshape(1, 1)


def kernel(y_s, y_s_adv, labels_s, y_t, y_t_adv, epoch):
    lab = labels_s.astype(jnp.int32)
    e4, e4i, t2, t2i, ce, cei, cs, csi = _sc_row_stats(y_s.T, y_s_adv.T, lab)

    lab_col = lab.reshape(_B, 1)
    lab_row = lab.reshape(1, _B)

    grid = (_B // _ROWS,)
    emp, disc = pl.pallas_call(
        _auc_kernel,
        grid=grid,
        in_specs=[
            pl.BlockSpec((_C, _B), lambda i: (0, 0)),
            pl.BlockSpec((_C, _B), lambda i: (0, 0)),
            pl.BlockSpec((_C, _B), lambda i: (0, 0)),
            pl.BlockSpec((_C, _B), lambda i: (0, 0)),
            pl.BlockSpec((_B, 1), lambda i: (0, 0)),
            pl.BlockSpec((_B, 1), lambda i: (0, 0)),
            pl.BlockSpec((_B, 1), lambda i: (0, 0)),
            pl.BlockSpec((_B, 1), lambda i: (0, 0)),
            pl.BlockSpec((_B, 1), lambda i: (0, 0)),
            pl.BlockSpec((1, _B), lambda i: (0, 0)),
        ],
        out_specs=[
            pl.BlockSpec((1, 1), lambda i: (0, 0)),
            pl.BlockSpec((1, 1), lambda i: (0, 0)),
        ],
        out_shape=[
            jax.ShapeDtypeStruct((1, 1), jnp.float32),
            jax.ShapeDtypeStruct((1, 1), jnp.float32),
        ],
        scratch_shapes=[
            pltpu.VMEM((_B, 1), jnp.float32),
            pltpu.VMEM((1, _B), jnp.float32),
            pltpu.VMEM((1, _B), jnp.float32),
        ],
    )(e4, e4i, t2, t2i, ce.reshape(_B, 1), cei.reshape(_B, 1),
      cs.reshape(_B, 1), csi.reshape(_B, 1), lab_col, lab_row)

    empirical = 0.25 * emp[0, 0]
    transfer = -0.5 * disc[0, 0]
    return (empirical, transfer)


# speedup vs baseline: 1.1470x; 1.0022x over previous
import functools
import math

import jax
import jax.numpy as jnp
from jax import lax
from jax.experimental import pallas as pl
from jax.experimental.pallas import tpu as pltpu
from jax.experimental.pallas import tpu_sc as plsc

_C = 10
_B = 2048
_EPS = 0.05
_ROWS = 256
_K0 = 1.0 + math.exp(2.0 * _EPS)
_LN2 = math.log(2.0)
_NC = 2
_NS = 16
_ACT = 16
_CHUNK = _B // _ACT


@functools.partial(
    pl.kernel,
    mesh=plsc.VectorSubcoreMesh(core_axis_name="c", subcore_axis_name="s"),
    out_type=[
        jax.ShapeDtypeStruct((_B,), jnp.float32),
        jax.ShapeDtypeStruct((_B,), jnp.float32),
        jax.ShapeDtypeStruct((_B,), jnp.float32),
        jax.ShapeDtypeStruct((_B,), jnp.float32),
    ],
    scratch_types=[
        pltpu.VMEM((_C, _CHUNK), jnp.float32),
        pltpu.VMEM((_C, _CHUNK), jnp.float32),
        pltpu.VMEM((_CHUNK,), jnp.int32),
        pltpu.VMEM((_CHUNK,), jnp.float32),
        pltpu.VMEM((_CHUNK,), jnp.float32),
        pltpu.VMEM((_CHUNK,), jnp.float32),
        pltpu.VMEM((_CHUNK,), jnp.float32),
    ],
)
def _sc_row_stats(ysT_hbm, ysaT_hbm, lab_hbm,
                  ce_hbm, cei_hbm, cs_hbm, csi_hbm,
                  ys_v, ysa_v, lab_v,
                  ce_v, cei_v, cs_v, csi_v):
    wid = lax.axis_index("s") * _NC + lax.axis_index("c")

    @pl.when(wid < _ACT)
    def _body():
        base = wid * _CHUNK
        csl = pl.ds(base, _CHUNK)

        pltpu.sync_copy(ysT_hbm.at[:, csl], ys_v)
        pltpu.sync_copy(ysaT_hbm.at[:, csl], ysa_v)
        pltpu.sync_copy(lab_hbm.at[csl], lab_v)

        def softmax_pick(v_ref, sl, lab_j):
            cols = [v_ref[c, sl] for c in range(_C)]
            m = cols[0]
            for c in range(1, _C):
                m = jnp.maximum(m, cols[c])
            es = [jnp.exp(cols[c] - m) for c in range(_C)]
            s = es[0]
            for c in range(1, _C):
                s = s + es[c]
            pick = jnp.zeros((16,), jnp.float32)
            for c in range(_C):
                pick = jnp.where(lab_j == c, es[c], pick)
            return pick / s

        for j in range(_CHUNK // 16):
            sl = pl.ds(16 * j, 16)
            lab_j = lab_v[sl]
            g = softmax_pick(ys_v, sl, lab_j)
            ga = softmax_pick(ysa_v, sl, lab_j)
            ce_v[sl] = jnp.exp((_EPS + 4.0) - 4.0 * g)
            cei_v[sl] = jnp.exp((_EPS - 4.0) + 4.0 * g)
            cs_v[sl] = jnp.exp(_EPS + 2.0 * (ga - g))
            csi_v[sl] = jnp.exp(_EPS - 2.0 * (ga - g))

        pltpu.sync_copy(ce_v, ce_hbm.at[csl])
        pltpu.sync_copy(cei_v, cei_hbm.at[csl])
        pltpu.sync_copy(cs_v, cs_hbm.at[csl])
        pltpu.sync_copy(csi_v, csi_hbm.at[csl])


def _auc_kernel(ys_ref, ysa_ref, ce_ref, cei_ref, cs_ref, csi_ref,
                labc_ref, labr_ref, emp_ref, disc_ref,
                e4p_ref, e4pi_ref, t2_ref, t2i_ref, w_ref,
                acc_e_ref, acc_s_ref):
    i = pl.program_id(0)
    nsteps = pl.num_programs(0)
    lab_row = labr_ref[...]

    @pl.when(i == 0)
    def _build_tables():
        def softmax(x):
            m = jnp.max(x, axis=1, keepdims=True)
            e = jnp.exp(x - m)
            return e / jnp.sum(e, axis=1, keepdims=True)
        p = softmax(ys_ref[...])
        pa = softmax(ysa_ref[...])
        e4p = jnp.exp(4.0 * p)
        t2 = jnp.exp(2.0 * (p - pa))
        e4p_ref[...] = e4p
        e4pi_ref[...] = 1.0 / e4p
        t2_ref[...] = t2
        t2i_ref[...] = 1.0 / t2
        lab_all = labc_ref[...]
        w = jnp.zeros((_B, 1), jnp.float32)
        for c in range(_C):
            n_c = jnp.sum((lab_row == c).astype(jnp.float32))
            fac_c = _LN2 / (n_c * (_B - n_c))
            w = w + jnp.where(lab_all == c, fac_c, 0.0)
        w_ref[...] = w
        acc_e_ref[...] = jnp.zeros((1, _B), jnp.float32)
        acc_s_ref[...] = jnp.zeros((1, _B), jnp.float32)

    rows = pl.ds(i * _ROWS, _ROWS)
    lab_blk = labc_ref[rows, :]

    cls = jax.lax.broadcasted_iota(jnp.int32, (1, _C), 1)
    onehot = (lab_blk == cls).astype(jnp.float32)

    dot = functools.partial(
        jax.lax.dot_general,
        dimension_numbers=(((1,), (1,)), ((), ())),
        preferred_element_type=jnp.float32,
        precision=jax.lax.Precision.DEFAULT,
    )
    h_e = dot(onehot * ce_ref[rows, :], e4p_ref[...])
    r_e = dot(onehot * cei_ref[rows, :], e4pi_ref[...])
    h_s = dot(onehot * cs_ref[rows, :], t2_ref[...])
    r_s = dot(onehot * csi_ref[rows, :], t2i_ref[...])

    wv = jnp.where(lab_blk != lab_row, w_ref[rows, :], 0.0)

    l_e = jnp.log2(_K0 + h_e + r_e)
    l_s = jnp.log2(_K0 + h_s + r_s)
    acc_e_ref[...] += jnp.sum(wv * l_e, axis=0, keepdims=True)
    acc_s_ref[...] += jnp.sum(wv * l_s, axis=0, keepdims=True)

    @pl.when(i == nsteps - 1)
    def _finish():
        emp_ref[...] = jnp.sum(acc_e_ref[...]).reshape(1, 1)
        disc_ref[...] = jnp.sum(acc_s_ref[...]).reshape(1, 1)


def kernel(y_s, y_s_adv, labels_s, y_t, y_t_adv, epoch):
    lab = labels_s.astype(jnp.int32)
    ce, cei, cs, csi = _sc_row_stats(y_s.T, y_s_adv.T, lab)

    lab_col = lab.reshape(_B, 1)
    lab_row = lab.reshape(1, _B)

    grid = (_B // _ROWS,)
    emp, disc = pl.pallas_call(
        _auc_kernel,
        grid=grid,
        in_specs=[
            pl.BlockSpec((_B, _C), lambda i: (0, 0)),
            pl.BlockSpec((_B, _C), lambda i: (0, 0)),
            pl.BlockSpec((_B, 1), lambda i: (0, 0)),
            pl.BlockSpec((_B, 1), lambda i: (0, 0)),
            pl.BlockSpec((_B, 1), lambda i: (0, 0)),
            pl.BlockSpec((_B, 1), lambda i: (0, 0)),
            pl.BlockSpec((_B, 1), lambda i: (0, 0)),
            pl.BlockSpec((1, _B), lambda i: (0, 0)),
        ],
        out_specs=[
            pl.BlockSpec((1, 1), lambda i: (0, 0)),
            pl.BlockSpec((1, 1), lambda i: (0, 0)),
        ],
        out_shape=[
            jax.ShapeDtypeStruct((1, 1), jnp.float32),
            jax.ShapeDtypeStruct((1, 1), jnp.float32),
        ],
        scratch_shapes=[
            pltpu.VMEM((_B, _C), jnp.float32),
            pltpu.VMEM((_B, _C), jnp.float32),
            pltpu.VMEM((_B, _C), jnp.float32),
            pltpu.VMEM((_B, _C), jnp.float32),
            pltpu.VMEM((_B, 1), jnp.float32),
            pltpu.VMEM((1, _B), jnp.float32),
            pltpu.VMEM((1, _B), jnp.float32),
        ],
    )(y_s, y_s_adv, ce.reshape(_B, 1), cei.reshape(_B, 1),
      cs.reshape(_B, 1), csi.reshape(_B, 1), lab_col, lab_row)

    empirical = 0.25 * emp[0, 0]
    transfer = -0.5 * disc[0, 0]
    return (empirical, transfer)


# speedup vs baseline: 1.9886x; 1.7338x over previous
import functools
import math

import jax
import jax.numpy as jnp
from jax.experimental import pallas as pl
from jax.experimental.pallas import tpu as pltpu

_C = 10
_B = 2048
_EPS = 0.05
_ROWS = 512
_K0 = 1.0 + math.exp(2.0 * _EPS)
_LN2 = math.log(2.0)


def _softmax(x):
    m = jnp.max(x, axis=1, keepdims=True)
    e = jnp.exp(x - m)
    return e / jnp.sum(e, axis=1, keepdims=True)


def _auc_kernel(ys_ref, ysa_ref, labc_ref, labr_ref, emp_ref, disc_ref,
                e4p_ref, e4pi_ref, t2_ref, t2i_ref, w_ref, acc_e_ref,
                acc_s_ref):
    i = pl.program_id(0)
    nsteps = pl.num_programs(0)
    lab_row = labr_ref[...]

    @pl.when(i == 0)
    def _build_tables():
        p = _softmax(ys_ref[...])
        pa = _softmax(ysa_ref[...])
        e4p = jnp.exp(4.0 * p)
        t2 = jnp.exp(2.0 * (p - pa))
        e4p_ref[...] = e4p
        e4pi_ref[...] = 1.0 / e4p
        t2_ref[...] = t2
        t2i_ref[...] = 1.0 / t2
        lab_all = labc_ref[...]
        w = jnp.zeros((_B, 1), jnp.float32)
        for c in range(_C):
            n_c = jnp.sum((lab_row == c).astype(jnp.float32))
            fac_c = _LN2 / (n_c * (_B - n_c))
            w = w + jnp.where(lab_all == c, fac_c, 0.0)
        w_ref[...] = w
        acc_e_ref[...] = jnp.zeros((1, _B), jnp.float32)
        acc_s_ref[...] = jnp.zeros((1, _B), jnp.float32)

    rows = pl.ds(i * _ROWS, _ROWS)
    lab_blk = labc_ref[rows, :]

    cls = jax.lax.broadcasted_iota(jnp.int32, (1, _C), 1)
    onehot = (lab_blk == cls).astype(jnp.float32)

    e4g = jnp.sum(onehot * e4p_ref[rows, :], axis=1, keepdims=True)
    t2g = jnp.sum(onehot * t2_ref[rows, :], axis=1, keepdims=True)
    c_e = math.exp(_EPS + 4.0) / e4g
    c_e_inv = math.exp(_EPS - 4.0) * e4g
    c_s = math.exp(_EPS) / t2g
    c_s_inv = math.exp(_EPS) * t2g

    dot = functools.partial(
        jax.lax.dot_general,
        dimension_numbers=(((1,), (1,)), ((), ())),
        preferred_element_type=jnp.float32,
        precision=jax.lax.Precision.DEFAULT,
    )
    h_e = dot(onehot * c_e, e4p_ref[...])
    r_e = dot(onehot * c_e_inv, e4pi_ref[...])
    h_s = dot(onehot * c_s, t2_ref[...])
    r_s = dot(onehot * c_s_inv, t2i_ref[...])

    wv = jnp.where(lab_blk != lab_row, w_ref[rows, :], 0.0)

    l_e = jnp.log2(_K0 + h_e + r_e)
    l_s = jnp.log2(_K0 + h_s + r_s)
    acc_e_ref[...] += jnp.sum(wv * l_e, axis=0, keepdims=True)
    acc_s_ref[...] += jnp.sum(wv * l_s, axis=0, keepdims=True)

    @pl.when(i == nsteps - 1)
    def _finish():
        emp_ref[...] = jnp.sum(acc_e_ref[...]).reshape(1, 1)
        disc_ref[...] = jnp.sum(acc_s_ref[...]).reshape(1, 1)


def kernel(y_s, y_s_adv, labels_s, y_t, y_t_adv, epoch):
    lab = labels_s.astype(jnp.int32)
    lab_col = lab.reshape(_B, 1)
    lab_row = lab.reshape(1, _B)

    grid = (_B // _ROWS,)
    emp, disc = pl.pallas_call(
        _auc_kernel,
        grid=grid,
        in_specs=[
            pl.BlockSpec((_B, _C), lambda i: (0, 0)),
            pl.BlockSpec((_B, _C), lambda i: (0, 0)),
            pl.BlockSpec((_B, 1), lambda i: (0, 0)),
            pl.BlockSpec((1, _B), lambda i: (0, 0)),
        ],
        out_specs=[
            pl.BlockSpec((1, 1), lambda i: (0, 0)),
            pl.BlockSpec((1, 1), lambda i: (0, 0)),
        ],
        out_shape=[
            jax.ShapeDtypeStruct((1, 1), jnp.float32),
            jax.ShapeDtypeStruct((1, 1), jnp.float32),
        ],
        scratch_shapes=[
            pltpu.VMEM((_B, _C), jnp.float32),
            pltpu.VMEM((_B, _C), jnp.float32),
            pltpu.VMEM((_B, _C), jnp.float32),
            pltpu.VMEM((_B, _C), jnp.float32),
            pltpu.VMEM((_B, 1), jnp.float32),
            pltpu.VMEM((1, _B), jnp.float32),
            pltpu.VMEM((1, _B), jnp.float32),
        ],
    )(y_s, y_s_adv, lab_col, lab_row)

    empirical = 0.25 * emp[0, 0]
    transfer = -0.5 * disc[0, 0]
    return (empirical, transfer)


# speedup vs baseline: 2.1440x; 1.0781x over previous
import functools
import math

import jax
import jax.numpy as jnp
from jax.experimental import pallas as pl
from jax.experimental.pallas import tpu as pltpu

_C = 10
_B = 2048
_EPS = 0.05
_ROWS = 256
_K0 = 1.0 + math.exp(2.0 * _EPS)
_LN2 = math.log(2.0)


def _softmax(x):
    m = jnp.max(x, axis=1, keepdims=True)
    e = jnp.exp(x - m)
    return e / jnp.sum(e, axis=1, keepdims=True)


def _auc_kernel(ys_ref, ysa_ref, labc_ref, labr_ref, emp_ref, disc_ref,
                ee_ref, ts_ref, red_ref, w_ref):
    i = pl.program_id(0)
    nsteps = pl.num_programs(0)
    lab_row = labr_ref[...]

    @pl.when(i == 0)
    def _build_tables():
        p = _softmax(ys_ref[...])
        pa = _softmax(ysa_ref[...])
        e4p = jnp.exp(4.0 * p)
        t2 = jnp.exp(2.0 * (p - pa))
        k0col = jnp.full((_B, 1), _K0, jnp.float32)
        ee_ref[...] = jnp.concatenate([e4p, 1.0 / e4p, k0col], axis=1)
        ts_ref[...] = jnp.concatenate([t2, 1.0 / t2, k0col], axis=1)
        lab_all = labc_ref[...]
        cls = jax.lax.broadcasted_iota(jnp.int32, (1, _C), 1)
        oh_full = (lab_all == cls).astype(jnp.float32)
        ones = jnp.ones((_B, 1), jnp.float32)
        red_ref[...] = jnp.concatenate([oh_full, ones], axis=1)
        w = jnp.zeros((_B, 1), jnp.float32)
        for c in range(_C):
            n_c = jnp.sum((lab_row == c).astype(jnp.float32))
            fac_c = _LN2 / (n_c * (_B - n_c))
            w = w + jnp.where(lab_all == c, fac_c, 0.0)
        w_ref[...] = w

    @pl.when(i == 0)
    def _init_out():
        emp_ref[...] = jnp.zeros((1, 1), jnp.float32)
        disc_ref[...] = jnp.zeros((1, 1), jnp.float32)

    rows = pl.ds(i * _ROWS, _ROWS)
    lab_blk = labc_ref[rows, :]

    cls = jax.lax.broadcasted_iota(jnp.int32, (1, _C), 1)
    onehot = (lab_blk == cls).astype(jnp.float32)

    e4g = jnp.sum(onehot * ee_ref[rows, 0:_C], axis=1, keepdims=True)
    t2g = jnp.sum(onehot * ts_ref[rows, 0:_C], axis=1, keepdims=True)
    c_e = math.exp(_EPS + 4.0) / e4g
    c_e_inv = math.exp(_EPS - 4.0) * e4g
    c_s = math.exp(_EPS) / t2g
    c_s_inv = math.exp(_EPS) * t2g

    ones_col = jnp.ones((_ROWS, 1), jnp.float32)
    oh_e = jnp.concatenate([onehot * c_e, onehot * c_e_inv, ones_col], axis=1)
    oh_s = jnp.concatenate([onehot * c_s, onehot * c_s_inv, ones_col], axis=1)

    dot = functools.partial(
        jax.lax.dot_general,
        dimension_numbers=(((1,), (1,)), ((), ())),
        preferred_element_type=jnp.float32,
        precision=jax.lax.Precision.DEFAULT,
    )
    l_e = jnp.log2(dot(oh_e, ee_ref[...]))
    l_s = jnp.log2(dot(oh_s, ts_ref[...]))

    dot_red = functools.partial(
        jax.lax.dot_general,
        dimension_numbers=(((1,), (0,)), ((), ())),
        preferred_element_type=jnp.float32,
        precision=jax.lax.Precision.DEFAULT,
    )
    s_e = dot_red(l_e, red_ref[...])
    s_s = dot_red(l_s, red_ref[...])
    w_blk = w_ref[rows, :]
    own_e = jnp.sum(onehot * s_e[:, 0:_C], axis=1, keepdims=True)
    own_s = jnp.sum(onehot * s_s[:, 0:_C], axis=1, keepdims=True)
    emp = jnp.sum(w_blk * (s_e[:, _C:_C + 1] - own_e)).reshape(1, 1)
    disc = jnp.sum(w_blk * (s_s[:, _C:_C + 1] - own_s)).reshape(1, 1)

    emp_ref[...] += emp
    disc_ref[...] += disc


def kernel(y_s, y_s_adv, labels_s, y_t, y_t_adv, epoch):
    lab = labels_s.astype(jnp.int32)
    lab_col = lab.reshape(_B, 1)
    lab_row = lab.reshape(1, _B)

    grid = (_B // _ROWS,)
    emp, disc = pl.pallas_call(
        _auc_kernel,
        grid=grid,
        in_specs=[
            pl.BlockSpec((_B, _C), lambda i: (0, 0)),
            pl.BlockSpec((_B, _C), lambda i: (0, 0)),
            pl.BlockSpec((_B, 1), lambda i: (0, 0)),
            pl.BlockSpec((1, _B), lambda i: (0, 0)),
        ],
        out_specs=[
            pl.BlockSpec((1, 1), lambda i: (0, 0)),
            pl.BlockSpec((1, 1), lambda i: (0, 0)),
        ],
        out_shape=[
            jax.ShapeDtypeStruct((1, 1), jnp.float32),
            jax.ShapeDtypeStruct((1, 1), jnp.float32),
        ],
        scratch_shapes=[
            pltpu.VMEM((_B, 2 * _C + 1), jnp.float32),
            pltpu.VMEM((_B, 2 * _C + 1), jnp.float32),
            pltpu.VMEM((_B, _C + 1), jnp.float32),
            pltpu.VMEM((_B, 1), jnp.float32),
        ],
    )(y_s, y_s_adv, lab_col, lab_row)

    empirical = 0.25 * emp[0, 0]
    transfer = -0.5 * disc[0, 0]
    return (empirical, transfer)
